# baseline (device time: 67102 ns/iter reference)
import jax
import jax.numpy as jnp
from jax import lax
from jax.experimental import pallas as pl
from jax.experimental.pallas import tpu as pltpu

HALF = 2048
Q = HALF // 2
D = 2048
CH = 32
NC = Q // CH


def kernel(partial, gamma):
    g = gamma.reshape(1, D)

    def body(p_ref, g_ref, out_ref, peer_stage, my_stage,
             ysend, yrecv, xsend, xrecv,
             stage_sems1, stage_sems2,
             ysend_sems, yrecv_sems, xsend_sems, xrecv_sems,
             out_sems1, out_sems2):
        my_x = lax.axis_index("x")
        my_y = lax.axis_index("y")
        peer_y = 1 - my_y
        peer_x = 1 - my_x

        barrier = pltpu.get_barrier_semaphore()
        pl.semaphore_signal(
            barrier, inc=1,
            device_id=(my_x, peer_y), device_id_type=pl.DeviceIdType.MESH)
        pl.semaphore_signal(
            barrier, inc=1,
            device_id=(peer_x, my_y), device_id_type=pl.DeviceIdType.MESH)
        pl.semaphore_wait(barrier, 2)

        st_send = []
        for c in range(NC):
            st = pltpu.make_async_copy(
                p_ref.at[0, pl.ds(peer_y * HALF + my_x * Q + c * CH, CH), :],
                peer_stage.at[pl.ds(c * CH, CH), :], stage_sems1.at[c])
            st.start()
            st_send.append(st)
        st_mine = []
        for c in range(NC):
            st = pltpu.make_async_copy(
                p_ref.at[0, pl.ds(my_y * HALF + my_x * Q + c * CH, CH), :],
                my_stage.at[pl.ds(c * CH, CH), :], stage_sems2.at[c])
            st.start()
            st_mine.append(st)

        y_rdmas = []
        for c in range(NC):
            st_send[c].wait()
            ysend[c] = peer_stage[pl.ds(c * CH, CH), :].astype(jnp.bfloat16)
            r = pltpu.make_async_remote_copy(
                src_ref=ysend.at[c], dst_ref=yrecv.at[c],
                send_sem=ysend_sems.at[c], recv_sem=yrecv_sems.at[c],
                device_id=(my_x, peer_y),
                device_id_type=pl.DeviceIdType.MESH)
            r.start()
            y_rdmas.append(r)

        x_rdmas = []
        out_dmas = []
        for c in range(NC):
            sl = pl.ds(c * CH, CH)
            st_mine[c].wait()
            y_rdmas[c].wait_recv()
            s = my_stage[sl, :] + yrecv[c].astype(jnp.float32)
            ms = jnp.mean(s * s, axis=-1, keepdims=True)
            res = s * lax.rsqrt(ms + 1e-6) * g_ref[0, :][None, :]
            xsend[c] = res.astype(jnp.bfloat16)
            r = pltpu.make_async_remote_copy(
                src_ref=xsend.at[c], dst_ref=xrecv.at[c],
                send_sem=xsend_sems.at[c], recv_sem=xrecv_sems.at[c],
                device_id=(peer_x, my_y),
                device_id_type=pl.DeviceIdType.MESH)
            r.start()
            x_rdmas.append(r)
            o = pltpu.make_async_copy(
                xsend.at[c],
                out_ref.at[pl.ds(my_x * Q + c * CH, CH), :],
                out_sems1.at[c])
            o.start()
            out_dmas.append(o)

        for c in range(NC):
            x_rdmas[c].wait_recv()
            o = pltpu.make_async_copy(
                xrecv.at[c],
                out_ref.at[pl.ds(peer_x * Q + c * CH, CH), :],
                out_sems2.at[c])
            o.start()
            out_dmas.append(o)

        for o in out_dmas:
            o.wait()
        for c in range(NC):
            y_rdmas[c].wait_send()
            x_rdmas[c].wait_send()

    return pl.pallas_call(
        body,
        out_shape=jax.ShapeDtypeStruct((HALF, D), jnp.bfloat16),
        in_specs=[
            pl.BlockSpec(memory_space=pl.ANY),
            pl.BlockSpec(memory_space=pltpu.VMEM),
        ],
        out_specs=pl.BlockSpec(memory_space=pl.ANY),
        scratch_shapes=[
            pltpu.VMEM((Q, D), jnp.float32),
            pltpu.VMEM((Q, D), jnp.float32),
            pltpu.VMEM((NC, CH, D), jnp.bfloat16),
            pltpu.VMEM((NC, CH, D), jnp.bfloat16),
            pltpu.VMEM((NC, CH, D), jnp.bfloat16),
            pltpu.VMEM((NC, CH, D), jnp.bfloat16),
            pltpu.SemaphoreType.DMA((NC,)),
            pltpu.SemaphoreType.DMA((NC,)),
            pltpu.SemaphoreType.DMA((NC,)),
            pltpu.SemaphoreType.DMA((NC,)),
            pltpu.SemaphoreType.DMA((NC,)),
            pltpu.SemaphoreType.DMA((NC,)),
            pltpu.SemaphoreType.DMA((NC,)),
            pltpu.SemaphoreType.DMA((NC,)),
        ],
        compiler_params=pltpu.CompilerParams(
            collective_id=0, vmem_limit_bytes=60 * 1024 * 1024
        ),
    )(partial, g)


# device time: 64886 ns/iter; 1.0342x vs baseline; 1.0342x over previous
import jax
import jax.numpy as jnp
from jax import lax
from jax.experimental import pallas as pl
from jax.experimental.pallas import tpu as pltpu

HALF = 2048
Q = HALF // 2
D = 2048
CHUNKS = [16, 48] + [64] * 14 + [48, 16]
OFFS = [sum(CHUNKS[:i]) for i in range(len(CHUNKS))]
NC = len(CHUNKS)
assert sum(CHUNKS) == Q


def kernel(partial, gamma):
    g = gamma.reshape(1, D)

    def body(p_ref, g_ref, out_ref, peer_stage, my_stage,
             ysend, yrecv, xsend, xrecv,
             stage_sems1, stage_sems2,
             ysend_sems, yrecv_sems, xsend_sems, xrecv_sems,
             out_sems1, out_sems2):
        my_x = lax.axis_index("x")
        my_y = lax.axis_index("y")
        peer_y = 1 - my_y
        peer_x = 1 - my_x

        barrier = pltpu.get_barrier_semaphore()
        pl.semaphore_signal(
            barrier, inc=1,
            device_id=(my_x, peer_y), device_id_type=pl.DeviceIdType.MESH)
        pl.semaphore_signal(
            barrier, inc=1,
            device_id=(peer_x, my_y), device_id_type=pl.DeviceIdType.MESH)
        pl.semaphore_wait(barrier, 2)

        st_send = []
        for c, (off, sz) in enumerate(zip(OFFS, CHUNKS)):
            st = pltpu.make_async_copy(
                p_ref.at[0, pl.ds(peer_y * HALF + my_x * Q + off, sz), :],
                peer_stage.at[pl.ds(off, sz), :], stage_sems1.at[c])
            st.start()
            st_send.append(st)
        st_mine = []
        for c, (off, sz) in enumerate(zip(OFFS, CHUNKS)):
            st = pltpu.make_async_copy(
                p_ref.at[0, pl.ds(my_y * HALF + my_x * Q + off, sz), :],
                my_stage.at[pl.ds(off, sz), :], stage_sems2.at[c])
            st.start()
            st_mine.append(st)

        y_rdmas = []
        for c, (off, sz) in enumerate(zip(OFFS, CHUNKS)):
            sl = pl.ds(off, sz)
            st_send[c].wait()
            ysend[sl, :] = peer_stage[sl, :].astype(jnp.bfloat16)
            r = pltpu.make_async_remote_copy(
                src_ref=ysend.at[sl, :], dst_ref=yrecv.at[sl, :],
                send_sem=ysend_sems.at[c], recv_sem=yrecv_sems.at[c],
                device_id=(my_x, peer_y),
                device_id_type=pl.DeviceIdType.MESH)
            r.start()
            y_rdmas.append(r)

        x_rdmas = []
        out_dmas = []
        for c, (off, sz) in enumerate(zip(OFFS, CHUNKS)):
            sl = pl.ds(off, sz)
            st_mine[c].wait()
            y_rdmas[c].wait_recv()
            s = my_stage[sl, :] + yrecv[sl, :].astype(jnp.float32)
            ms = jnp.mean(s * s, axis=-1, keepdims=True)
            res = s * lax.rsqrt(ms + 1e-6) * g_ref[0, :][None, :]
            xsend[sl, :] = res.astype(jnp.bfloat16)
            r = pltpu.make_async_remote_copy(
                src_ref=xsend.at[sl, :], dst_ref=xrecv.at[sl, :],
                send_sem=xsend_sems.at[c], recv_sem=xrecv_sems.at[c],
                device_id=(peer_x, my_y),
                device_id_type=pl.DeviceIdType.MESH)
            r.start()
            x_rdmas.append(r)
            o = pltpu.make_async_copy(
                xsend.at[sl, :],
                out_ref.at[pl.ds(my_x * Q + off, sz), :],
                out_sems1.at[c])
            o.start()
            out_dmas.append(o)

        for c, (off, sz) in enumerate(zip(OFFS, CHUNKS)):
            sl = pl.ds(off, sz)
            x_rdmas[c].wait_recv()
            o = pltpu.make_async_copy(
                xrecv.at[sl, :],
                out_ref.at[pl.ds(peer_x * Q + off, sz), :],
                out_sems2.at[c])
            o.start()
            out_dmas.append(o)

        for o in out_dmas:
            o.wait()
        for c in range(NC):
            y_rdmas[c].wait_send()
            x_rdmas[c].wait_send()

    return pl.pallas_call(
        body,
        out_shape=jax.ShapeDtypeStruct((HALF, D), jnp.bfloat16),
        in_specs=[
            pl.BlockSpec(memory_space=pl.ANY),
            pl.BlockSpec(memory_space=pltpu.VMEM),
        ],
        out_specs=pl.BlockSpec(memory_space=pl.ANY),
        scratch_shapes=[
            pltpu.VMEM((Q, D), jnp.float32),
            pltpu.VMEM((Q, D), jnp.float32),
            pltpu.VMEM((Q, D), jnp.bfloat16),
            pltpu.VMEM((Q, D), jnp.bfloat16),
            pltpu.VMEM((Q, D), jnp.bfloat16),
            pltpu.VMEM((Q, D), jnp.bfloat16),
            pltpu.SemaphoreType.DMA((NC,)),
            pltpu.SemaphoreType.DMA((NC,)),
            pltpu.SemaphoreType.DMA((NC,)),
            pltpu.SemaphoreType.DMA((NC,)),
            pltpu.SemaphoreType.DMA((NC,)),
            pltpu.SemaphoreType.DMA((NC,)),
            pltpu.SemaphoreType.DMA((NC,)),
            pltpu.SemaphoreType.DMA((NC,)),
        ],
        compiler_params=pltpu.CompilerParams(
            collective_id=0, vmem_limit_bytes=60 * 1024 * 1024
        ),
    )(partial, g)
